# baseline (speedup 1.0000x reference)
import functools

import jax
import jax.numpy as jnp
from jax import lax
from jax.experimental import pallas as pl
from jax.experimental.pallas import tpu as pltpu
from jax.experimental.pallas import tpu_sc as plsc

NC, NS = 2, 16
NW = NC * NS
L = 16


def _sc_add(x, pe):
    B, S, D = x.shape
    s_per_w = S // NW
    CH = 4
    n_chunks = s_per_w // CH

    @functools.partial(
        pl.kernel,
        out_type=jax.ShapeDtypeStruct((B, S, D), jnp.float32),
        mesh=plsc.VectorSubcoreMesh(
            core_axis_name="c", subcore_axis_name="s",
            num_cores=NC, num_subcores=NS),
        scratch_types=[
            pltpu.VMEM((2, CH, D), jnp.float32),
            pltpu.VMEM((2, B, CH, D), jnp.float32),
            pltpu.VMEM((2, B, CH, D), jnp.float32),
            pltpu.SemaphoreType.DMA,
            pltpu.SemaphoreType.DMA,
            pltpu.SemaphoreType.DMA,
            pltpu.SemaphoreType.DMA,
        ],
    )
    def sc_add(x_hbm, pe_hbm, out_hbm, pe_v, x_v, o_v, in0, in1, ou0, ou1):
        wid = lax.axis_index("s") * NC + lax.axis_index("c")
        base = wid * s_per_w
        in_sems = (in0, in1)
        out_sems = (ou0, ou1)

        def in_copies(c, slot):
            s0 = base + c * CH
            pltpu.async_copy(pe_hbm.at[pl.ds(s0, CH)], pe_v.at[slot],
                             in_sems[slot])
            pltpu.async_copy(x_hbm.at[:, pl.ds(s0, CH), :],
                             x_v.at[slot], in_sems[slot])

        def wait_in(slot):
            pltpu.make_async_copy(pe_hbm.at[pl.ds(base, CH)], pe_v.at[slot],
                                  in_sems[slot]).wait()
            pltpu.make_async_copy(x_hbm.at[:, pl.ds(base, CH), :],
                                  x_v.at[slot], in_sems[slot]).wait()

        def out_copies(c, slot):
            s0 = base + c * CH
            pltpu.async_copy(o_v.at[slot],
                             out_hbm.at[:, pl.ds(s0, CH), :],
                             out_sems[slot])

        def wait_out(slot):
            pltpu.make_async_copy(o_v.at[slot],
                                  out_hbm.at[:, pl.ds(base, CH), :],
                                  out_sems[slot]).wait()

        def compute(slot):
            def pos_body(si, carry):
                for dcol in range(D // L):
                    sl = pl.ds(dcol * L, L)
                    vec = pe_v[slot, si, sl]
                    for b in range(B):
                        o_v[slot, b, si, sl] = x_v[slot, b, si, sl] + vec
                return carry
            lax.fori_loop(0, CH, pos_body, 0)

        in_copies(0, 0)
        in_copies(1, 1)

        def loop_body(i2, carry):
            for slot in (0, 1):
                c = i2 * 2 + slot
                wait_in(slot)

                @pl.when(i2 > 0)
                def _():
                    wait_out(slot)

                compute(slot)
                out_copies(c, slot)

                @pl.when(c + 2 < n_chunks)
                def _():
                    in_copies(c + 2, slot)
            return carry

        lax.fori_loop(0, n_chunks // 2, loop_body, 0)
        wait_out(0)
        wait_out(1)

    return sc_add(x, pe)


def _tc_body(x_ref, pe_ref, o_ref):
    o_ref[...] = x_ref[...] + pe_ref[...][None, :, :]


def _tc_add(x, pe):
    B, S, D = x.shape
    BS = 512
    return pl.pallas_call(
        _tc_body,
        grid=(S // BS,),
        in_specs=[
            pl.BlockSpec((B, BS, D), lambda i: (0, i, 0)),
            pl.BlockSpec((BS, D), lambda i: (i, 0)),
        ],
        out_specs=pl.BlockSpec((B, BS, D), lambda i: (0, i, 0)),
        out_shape=jax.ShapeDtypeStruct((B, S, D), x.dtype),
    )(x, pe)


def kernel(x, pe):
    B = x.shape[0]
    b_sc = B // 2
    out_sc = _sc_add(x[:b_sc], pe)
    out_tc = _tc_add(x[b_sc:], pe)
    return jnp.concatenate([out_sc, out_tc], axis=0)


# speedup vs baseline: 1.3304x; 1.3304x over previous
import functools

import jax
import jax.numpy as jnp
from jax import lax
from jax.experimental import pallas as pl
from jax.experimental.pallas import tpu as pltpu
from jax.experimental.pallas import tpu_sc as plsc

NC, NS = 2, 16
NW = NC * NS
L = 16


def _sc_add(x, pe, b_sc):
    _, S, D = x.shape
    B = b_sc
    s_per_w = S // NW
    CH = 4
    n_chunks = s_per_w // CH

    @functools.partial(
        pl.kernel,
        out_type=jax.ShapeDtypeStruct((B, S, D), jnp.float32),
        mesh=plsc.VectorSubcoreMesh(
            core_axis_name="c", subcore_axis_name="s",
            num_cores=NC, num_subcores=NS),
        scratch_types=[
            pltpu.VMEM((2, CH, D), jnp.float32),
            pltpu.VMEM((2, B, CH, D), jnp.float32),
            pltpu.VMEM((2, B, CH, D), jnp.float32),
            pltpu.SemaphoreType.DMA,
            pltpu.SemaphoreType.DMA,
            pltpu.SemaphoreType.DMA,
            pltpu.SemaphoreType.DMA,
        ],
    )
    def sc_add(x_hbm, pe_hbm, out_hbm, pe_v, x_v, o_v, in0, in1, ou0, ou1):
        wid = lax.axis_index("s") * NC + lax.axis_index("c")
        base = wid * s_per_w
        in_sems = (in0, in1)
        out_sems = (ou0, ou1)

        def in_copies(c, slot):
            s0 = base + c * CH
            pltpu.async_copy(pe_hbm.at[pl.ds(s0, CH)], pe_v.at[slot],
                             in_sems[slot])
            pltpu.async_copy(x_hbm.at[pl.ds(0, B), pl.ds(s0, CH), :],
                             x_v.at[slot], in_sems[slot])

        def wait_in(slot):
            pltpu.make_async_copy(pe_hbm.at[pl.ds(base, CH)], pe_v.at[slot],
                                  in_sems[slot]).wait()
            pltpu.make_async_copy(x_hbm.at[pl.ds(0, B), pl.ds(base, CH), :],
                                  x_v.at[slot], in_sems[slot]).wait()

        def out_copies(c, slot):
            s0 = base + c * CH
            pltpu.async_copy(o_v.at[slot],
                             out_hbm.at[:, pl.ds(s0, CH), :],
                             out_sems[slot])

        def wait_out(slot):
            pltpu.make_async_copy(o_v.at[slot],
                                  out_hbm.at[:, pl.ds(base, CH), :],
                                  out_sems[slot]).wait()

        def compute(slot):
            def pos_body(si, carry):
                for dcol in range(D // L):
                    sl = pl.ds(dcol * L, L)
                    vec = pe_v[slot, si, sl]
                    for b in range(B):
                        o_v[slot, b, si, sl] = x_v[slot, b, si, sl] + vec
                return carry
            lax.fori_loop(0, CH, pos_body, 0)

        in_copies(0, 0)
        in_copies(1, 1)

        def loop_body(i2, carry):
            for slot in (0, 1):
                c = i2 * 2 + slot
                wait_in(slot)

                @pl.when(i2 > 0)
                def _():
                    wait_out(slot)

                compute(slot)
                out_copies(c, slot)

                @pl.when(c + 2 < n_chunks)
                def _():
                    in_copies(c + 2, slot)
            return carry

        lax.fori_loop(0, n_chunks // 2, loop_body, 0)
        wait_out(0)
        wait_out(1)

    return sc_add(x, pe)


def _tc_body(x_ref, pe_ref, o_ref):
    o_ref[...] = x_ref[...] + pe_ref[...][None, :, :]


def _tc_add(x, pe, b_sc):
    B, S, D = x.shape
    n_tc = B - b_sc
    BS = 512
    return pl.pallas_call(
        _tc_body,
        grid=(S // BS,),
        in_specs=[
            pl.BlockSpec((n_tc, BS, D), lambda i: (b_sc // n_tc, i, 0)),
            pl.BlockSpec((BS, D), lambda i: (i, 0)),
        ],
        out_specs=pl.BlockSpec((n_tc, BS, D), lambda i: (0, i, 0)),
        out_shape=jax.ShapeDtypeStruct((n_tc, S, D), x.dtype),
    )(x, pe)


def kernel(x, pe):
    B = x.shape[0]
    b_sc = B // 2
    out_sc = _sc_add(x, pe, b_sc)
    out_tc = _tc_add(x, pe, b_sc)
    return jnp.concatenate([out_sc, out_tc], axis=0)


# speedup vs baseline: 1.9641x; 1.4764x over previous
import functools

import jax
import jax.numpy as jnp
from jax import lax
from jax.experimental import pallas as pl
from jax.experimental.pallas import tpu as pltpu
from jax.experimental.pallas import tpu_sc as plsc

NC, NS = 2, 16
NW = NC * NS
L = 16
NBUF = 4


def kernel(x, pe):
    B, S, D = x.shape
    s_per_w = S // NW
    CH = 4
    n_chunks = s_per_w // CH

    @functools.partial(
        pl.kernel,
        out_type=jax.ShapeDtypeStruct((B, S, D), jnp.float32),
        mesh=plsc.VectorSubcoreMesh(
            core_axis_name="c", subcore_axis_name="s",
            num_cores=NC, num_subcores=NS),
        scratch_types=[
            pltpu.VMEM((NBUF, CH, D), jnp.float32),
            pltpu.VMEM((NBUF, B, CH, D), jnp.float32),
            pltpu.SemaphoreType.DMA,
            pltpu.SemaphoreType.DMA,
            pltpu.SemaphoreType.DMA,
            pltpu.SemaphoreType.DMA,
            pltpu.SemaphoreType.DMA,
            pltpu.SemaphoreType.DMA,
            pltpu.SemaphoreType.DMA,
            pltpu.SemaphoreType.DMA,
        ],
    )
    def sc_add(x_hbm, pe_hbm, out_hbm, pe_v, x_v,
               in0, in1, in2, in3, ou0, ou1, ou2, ou3):
        wid = lax.axis_index("s") * NC + lax.axis_index("c")
        base = wid * s_per_w
        in_sems = (in0, in1, in2, in3)
        out_sems = (ou0, ou1, ou2, ou3)

        def in_copies(c, slot):
            s0 = base + c * CH
            pltpu.async_copy(pe_hbm.at[pl.ds(s0, CH)], pe_v.at[slot],
                             in_sems[slot])
            pltpu.async_copy(x_hbm.at[:, pl.ds(s0, CH), :],
                             x_v.at[slot], in_sems[slot])

        def wait_in(slot):
            pltpu.make_async_copy(pe_hbm.at[pl.ds(base, CH)], pe_v.at[slot],
                                  in_sems[slot]).wait()
            pltpu.make_async_copy(x_hbm.at[:, pl.ds(base, CH), :],
                                  x_v.at[slot], in_sems[slot]).wait()

        def out_copies(c, slot):
            s0 = base + c * CH
            pltpu.async_copy(x_v.at[slot],
                             out_hbm.at[:, pl.ds(s0, CH), :],
                             out_sems[slot])

        def wait_out(slot):
            pltpu.make_async_copy(x_v.at[slot],
                                  out_hbm.at[:, pl.ds(base, CH), :],
                                  out_sems[slot]).wait()

        def compute(slot):
            def pos_body(si, carry):
                for dcol in range(D // L):
                    sl = pl.ds(dcol * L, L)
                    vec = pe_v[slot, si, sl]
                    for b in range(B):
                        plsc.addupdate(x_v.at[slot, b, si, sl], vec)
                return carry
            lax.fori_loop(0, CH, pos_body, 0)

        for k in range(NBUF):
            in_copies(k, k)

        def loop_body(i4, carry):
            for k in range(NBUF):
                c = i4 * NBUF + k
                wait_in(k)
                compute(k)
                out_copies(c, k)
                j = (k + 2) % NBUF

                @pl.when(jnp.logical_and(c >= 2, c + 2 < n_chunks))
                def _():
                    wait_out(j)
                    in_copies(c + 2, j)
            return carry

        lax.fori_loop(0, n_chunks // NBUF, loop_body, 0)
        for k in range(NBUF):
            wait_out(k)

    return sc_add(x, pe)
